# Initial kernel scaffold; baseline (speedup 1.0000x reference)
import functools

import jax
import jax.numpy as jnp
from jax import lax
from jax.experimental import pallas as pl
from jax.experimental.pallas import tpu as pltpu
from jax.experimental.pallas import tpu_sc as plsc

N_NODES = 100000
N_EDGES = 6400000

NW = 32
EPW = N_EDGES // NW
B = 2000
NCHUNK = EPW // B
G = 80
NG = B // G
NV = B // 16

_HALF = 0.5
_THREEHALF = 1.5
_MAGIC = jnp.int32(0x5F3759DF)


def _norm16(dx, dy, dz):
    sq = dx * dx + dy * dy + dz * dz
    y = plsc.bitcast(_MAGIC - (plsc.bitcast(sq, jnp.int32) >> 1), jnp.float32)
    y = y * (_THREEHALF - _HALF * sq * y * y)
    y = y * (_THREEHALF - _HALF * sq * y * y)
    return jnp.where(sq > 0.0, sq * y, 0.0)


def _sc_body(pos_hbm, edge_hbm, w_hbm, vec_hbm,
             sx, sy, sz, si, di, gsx, gsy, gsz, gdx, gdy, gdz,
             vec_v, w_v, sem):
    cid = lax.axis_index("c")
    sid = lax.axis_index("s")
    wid = cid * 16 + sid

    @pl.when(sid == 0)
    def _():
        pltpu.sync_copy(pos_hbm.at[0], sx)

    @pl.when(sid == 1)
    def _():
        pltpu.sync_copy(pos_hbm.at[1], sy)

    @pl.when(sid == 2)
    def _():
        pltpu.sync_copy(pos_hbm.at[2], sz)

    plsc.subcore_barrier()

    lanes = lax.iota(jnp.int32, 16)

    def chunk_body(k, _):
        base = wid * EPW + k * B
        pltpu.sync_copy(edge_hbm.at[0, pl.ds(base, B)], si)
        pltpu.sync_copy(edge_hbm.at[1, pl.ds(base, B)], di)

        def gather_body(j, _):
            s = pl.ds(j * G, G)
            cps = [
                pltpu.async_copy(sx.at[si.at[s]], gsx.at[s], sem),
                pltpu.async_copy(sy.at[si.at[s]], gsy.at[s], sem),
                pltpu.async_copy(sz.at[si.at[s]], gsz.at[s], sem),
                pltpu.async_copy(sx.at[di.at[s]], gdx.at[s], sem),
                pltpu.async_copy(sy.at[di.at[s]], gdy.at[s], sem),
                pltpu.async_copy(sz.at[di.at[s]], gdz.at[s], sem),
            ]
            for cp in cps:
                cp.wait()
            return 0

        lax.fori_loop(0, NG, gather_body, 0)

        def compute_body(i, _):
            s = pl.ds(i * 16, 16)
            dx = gsx[s] - gdx[s]
            dy = gsy[s] - gdy[s]
            dz = gsz[s] - gdz[s]
            w_v[s] = _norm16(dx, dy, dz)
            rows = i * 16 + lanes
            plsc.store_scatter(vec_v, [rows, jnp.zeros((16,), jnp.int32)], dx)
            plsc.store_scatter(vec_v, [rows, jnp.ones((16,), jnp.int32)], dy)
            plsc.store_scatter(vec_v, [rows, jnp.full((16,), 2, jnp.int32)], dz)
            return 0

        lax.fori_loop(0, NV, compute_body, 0)

        pltpu.sync_copy(w_v, w_hbm.at[pl.ds(base, B)])
        pltpu.sync_copy(vec_v, vec_hbm.at[pl.ds(base, B)])
        return 0

    lax.fori_loop(0, NCHUNK, chunk_body, 0)


@jax.jit
def _distance_sc(pos_planar, edge_index):
    mesh = plsc.VectorSubcoreMesh(core_axis_name="c", subcore_axis_name="s")
    kfn = pl.kernel(
        _sc_body,
        out_type=[
            jax.ShapeDtypeStruct((N_EDGES,), jnp.float32),
            jax.ShapeDtypeStruct((N_EDGES, 3), jnp.float32),
        ],
        mesh=mesh,
        scratch_types=[
            pltpu.VMEM_SHARED((N_NODES,), jnp.float32),
            pltpu.VMEM_SHARED((N_NODES,), jnp.float32),
            pltpu.VMEM_SHARED((N_NODES,), jnp.float32),
            pltpu.VMEM((B,), jnp.int32),
            pltpu.VMEM((B,), jnp.int32),
            pltpu.VMEM((B,), jnp.float32),
            pltpu.VMEM((B,), jnp.float32),
            pltpu.VMEM((B,), jnp.float32),
            pltpu.VMEM((B,), jnp.float32),
            pltpu.VMEM((B,), jnp.float32),
            pltpu.VMEM((B,), jnp.float32),
            pltpu.VMEM((B, 3), jnp.float32),
            pltpu.VMEM((B,), jnp.float32),
            pltpu.SemaphoreType.DMA,
        ],
    )
    return kfn(pos_planar, edge_index)


def kernel(pos, edge_index):
    pos_planar = pos.T
    edge_weight, edge_vec = _distance_sc(pos_planar, edge_index)
    return (edge_index, edge_weight, edge_vec)


# speedup vs baseline: 9.9752x; 9.9752x over previous
import functools

import jax
import jax.numpy as jnp
from jax import lax
from jax.experimental import pallas as pl
from jax.experimental.pallas import tpu as pltpu
from jax.experimental.pallas import tpu_sc as plsc

N_NODES = 100000
N_EDGES = 6400000

NW = 32
EPW = N_EDGES // NW
B = 2000
NCHUNK = EPW // B
G = 80
NG = B // G
NV = B // 16

_HALF = 0.5
_THREEHALF = 1.5
_MAGIC = 0x5F3759DF


def _norm16(dx, dy, dz):
    sq = dx * dx + dy * dy + dz * dz
    magic = jnp.full((16,), _MAGIC, jnp.int32)
    y = lax.bitcast_convert_type(
        magic - (lax.bitcast_convert_type(sq, jnp.int32) >> 1), jnp.float32)
    y = y * (_THREEHALF - _HALF * sq * y * y)
    y = y * (_THREEHALF - _HALF * sq * y * y)
    return jnp.where(sq > 0.0, sq * y, 0.0)


_NSTAGE = 10
_SSLICE = N_NODES // _NSTAGE


def _sc_body(pos_hbm, edge_hbm, w_hbm, vec_hbm,
             sx, sy, sz, si, di, gsx, gsy, gsz, gdx, gdy, gdz,
             vec_v, w_v, stage_v, sem):
    cid = lax.axis_index("c")
    sid = lax.axis_index("s")
    wid = cid * 16 + sid

    for c, comp in enumerate((sx, sy, sz)):
        for j in range(_NSTAGE):
            t = c * _NSTAGE + j

            @pl.when(sid == t % 16)
            def _(c=c, comp=comp, j=j):
                off = j * _SSLICE
                pltpu.sync_copy(
                    pos_hbm.at[pl.ds(c * N_NODES + off, _SSLICE)], stage_v)
                pltpu.sync_copy(stage_v, comp.at[pl.ds(off, _SSLICE)])

    plsc.subcore_barrier()

    lanes = lax.iota(jnp.int32, 16)

    def chunk_body(k, _):
        base = wid * EPW + k * B
        pltpu.sync_copy(edge_hbm.at[pl.ds(base, B)], si)
        pltpu.sync_copy(edge_hbm.at[pl.ds(N_EDGES + base, B)], di)

        def gather_body(j, _):
            s = pl.ds(j * G, G)
            cps = [
                pltpu.async_copy(sx.at[si.at[s]], gsx.at[s], sem),
                pltpu.async_copy(sy.at[si.at[s]], gsy.at[s], sem),
                pltpu.async_copy(sz.at[si.at[s]], gsz.at[s], sem),
                pltpu.async_copy(sx.at[di.at[s]], gdx.at[s], sem),
                pltpu.async_copy(sy.at[di.at[s]], gdy.at[s], sem),
                pltpu.async_copy(sz.at[di.at[s]], gdz.at[s], sem),
            ]
            for cp in cps:
                cp.wait()
            return 0

        lax.fori_loop(0, NG, gather_body, 0)

        def compute_body(i, _):
            s = pl.ds(i * 16, 16)
            dx = gsx[s] - gdx[s]
            dy = gsy[s] - gdy[s]
            dz = gsz[s] - gdz[s]
            w_v[s] = _norm16(dx, dy, dz)
            flat = i * 48 + 3 * lanes
            plsc.store_scatter(vec_v, [flat], dx)
            plsc.store_scatter(vec_v, [flat + 1], dy)
            plsc.store_scatter(vec_v, [flat + 2], dz)
            return 0

        lax.fori_loop(0, NV, compute_body, 0)

        pltpu.sync_copy(w_v, w_hbm.at[pl.ds(base, B)])
        pltpu.sync_copy(vec_v, vec_hbm.at[pl.ds(base * 3, B * 3)])
        return 0

    lax.fori_loop(0, NCHUNK, chunk_body, 0)


@jax.jit
def _distance_sc(pos_flat, edge_flat):
    mesh = plsc.VectorSubcoreMesh(core_axis_name="c", subcore_axis_name="s")
    kfn = pl.kernel(
        _sc_body,
        out_type=[
            jax.ShapeDtypeStruct((N_EDGES,), jnp.float32),
            jax.ShapeDtypeStruct((N_EDGES * 3,), jnp.float32),
        ],
        mesh=mesh,
        compiler_params=pltpu.CompilerParams(needs_layout_passes=False),
        scratch_types=[
            pltpu.VMEM_SHARED((N_NODES,), jnp.float32),
            pltpu.VMEM_SHARED((N_NODES,), jnp.float32),
            pltpu.VMEM_SHARED((N_NODES,), jnp.float32),
            pltpu.VMEM((B,), jnp.int32),
            pltpu.VMEM((B,), jnp.int32),
            pltpu.VMEM((B,), jnp.float32),
            pltpu.VMEM((B,), jnp.float32),
            pltpu.VMEM((B,), jnp.float32),
            pltpu.VMEM((B,), jnp.float32),
            pltpu.VMEM((B,), jnp.float32),
            pltpu.VMEM((B,), jnp.float32),
            pltpu.VMEM((B * 3,), jnp.float32),
            pltpu.VMEM((B,), jnp.float32),
            pltpu.VMEM((_SSLICE,), jnp.float32),
            pltpu.SemaphoreType.DMA,
        ],
    )
    return kfn(pos_flat, edge_flat)


def kernel(pos, edge_index):
    pos_flat = pos.T.reshape(3 * N_NODES)
    edge_flat = edge_index.reshape(2 * N_EDGES)
    edge_weight, edge_vec_flat = _distance_sc(pos_flat, edge_flat)
    return (edge_index, edge_weight, edge_vec_flat.reshape(N_EDGES, 3))


# speedup vs baseline: 10.6180x; 1.0644x over previous
import jax
import jax.numpy as jnp
from jax import lax
from jax.experimental import pallas as pl
from jax.experimental.pallas import tpu as pltpu
from jax.experimental.pallas import tpu_sc as plsc

N_NODES = 100000
N_EDGES = 6400000

NW = 32
EPW = N_EDGES // NW
B = 2000
NCHUNK = EPW // B
NV = B // 16

_HALF = 0.5
_THREEHALF = 1.5
_MAGIC = 0x5F3759DF


def _norm16(dx, dy, dz):
    sq = dx * dx + dy * dy + dz * dz
    magic = jnp.full((16,), _MAGIC, jnp.int32)
    y = lax.bitcast_convert_type(
        magic - (lax.bitcast_convert_type(sq, jnp.int32) >> 1), jnp.float32)
    y = y * (_THREEHALF - _HALF * sq * y * y)
    y = y * (_THREEHALF - _HALF * sq * y * y)
    return jnp.where(sq > 0.0, sq * y, 0.0)


def _sc_body(pos_hbm, edge_hbm, w_hbm, vec_hbm,
             si0, di0, si1, di1,
             gs0, gd0, gs1, gd1,
             vec0, vec1, w0, w1,
             gsem0, gsem1, osem0, osem1):
    cid = lax.axis_index("c")
    sid = lax.axis_index("s")
    wid = cid * 16 + sid

    lanes = lax.iota(jnp.int32, 16)
    lanes3 = 3 * lanes
    lanes4 = 4 * lanes
    zeros16 = jnp.zeros((16,), jnp.int32)
    bufs = ((si0, di0, gs0, gd0, vec0, w0, gsem0, osem0),
            (si1, di1, gs1, gd1, vec1, w1, gsem1, osem1))

    def load_idx(k, si, di):
        base = wid * EPW + k * B
        pltpu.sync_copy(edge_hbm.at[pl.ds(base, B)], si)
        pltpu.sync_copy(edge_hbm.at[pl.ds(N_EDGES + base, B)], di)

    def fire_gathers(si, di, gs, gd, gsem):
        pltpu.async_copy(pos_hbm.at[si], gs, gsem)
        pltpu.async_copy(pos_hbm.at[di], gd, gsem)

    def wait_gathers(si, di, gs, gd, gsem):
        pltpu.make_async_copy(pos_hbm.at[si], gs, gsem).wait()
        pltpu.make_async_copy(pos_hbm.at[di], gd, gsem).wait()

    def compute(gs, gd, vec_v, w_v):
        def body(i, _):
            rows = i * 16 + lanes
            sxv = plsc.load_gather(gs, [rows, zeros16])
            syv = plsc.load_gather(gs, [rows, zeros16 + 1])
            szv = plsc.load_gather(gs, [rows, zeros16 + 2])
            dxv = plsc.load_gather(gd, [rows, zeros16])
            dyv = plsc.load_gather(gd, [rows, zeros16 + 1])
            dzv = plsc.load_gather(gd, [rows, zeros16 + 2])
            dx = sxv - dxv
            dy = syv - dyv
            dz = szv - dzv
            w_v[pl.ds(i * 16, 16)] = _norm16(dx, dy, dz)
            flat = i * 48 + lanes3
            plsc.store_scatter(vec_v, [flat], dx)
            plsc.store_scatter(vec_v, [flat + 1], dy)
            plsc.store_scatter(vec_v, [flat + 2], dz)
            return 0

        lax.fori_loop(0, NV, body, 0)

    def fire_out(k, vec_v, w_v, osem):
        base = wid * EPW + k * B
        pltpu.async_copy(vec_v, vec_hbm.at[pl.ds(base * 3, B * 3)], osem)
        pltpu.async_copy(w_v, w_hbm.at[pl.ds(base, B)], osem)

    def wait_out(k, vec_v, w_v, osem):
        base = wid * EPW + k * B
        pltpu.make_async_copy(
            vec_v, vec_hbm.at[pl.ds(base * 3, B * 3)], osem).wait()
        pltpu.make_async_copy(w_v, w_hbm.at[pl.ds(base, B)], osem).wait()

    load_idx(0, si0, di0)
    fire_gathers(si0, di0, gs0, gd0, gsem0)

    def outer(ki, _):
        for h in (0, 1):
            k = 2 * ki + h
            si, di, gs, gd, vec_v, w_v, gsem, osem = bufs[h]
            nsi, ndi, ngs, ngd, _nv, _nw, ngsem, _no = bufs[1 - h]

            wait_gathers(si, di, gs, gd, gsem)

            @pl.when(k + 1 < NCHUNK)
            def _():
                load_idx(k + 1, nsi, ndi)
                fire_gathers(nsi, ndi, ngs, ngd, ngsem)

            @pl.when(ki >= 1)
            def _():
                wait_out(k, vec_v, w_v, osem)

            compute(gs, gd, vec_v, w_v)
            fire_out(k, vec_v, w_v, osem)
        return 0

    lax.fori_loop(0, NCHUNK // 2, outer, 0)

    for h in (0, 1):
        _si, _di, _gs, _gd, vec_v, w_v, _gsem, osem = bufs[h]
        wait_out(0, vec_v, w_v, osem)


@jax.jit
def _distance_sc(pos4, edge_flat):
    mesh = plsc.VectorSubcoreMesh(core_axis_name="c", subcore_axis_name="s")
    kfn = pl.kernel(
        _sc_body,
        out_type=[
            jax.ShapeDtypeStruct((N_EDGES,), jnp.float32),
            jax.ShapeDtypeStruct((N_EDGES * 3,), jnp.float32),
        ],
        mesh=mesh,
        compiler_params=pltpu.CompilerParams(
            needs_layout_passes=False, use_tc_tiling_on_sc=False),
        scratch_types=[
            pltpu.VMEM((B,), jnp.int32),
            pltpu.VMEM((B,), jnp.int32),
            pltpu.VMEM((B,), jnp.int32),
            pltpu.VMEM((B,), jnp.int32),
            pltpu.VMEM((B, 4), jnp.float32),
            pltpu.VMEM((B, 4), jnp.float32),
            pltpu.VMEM((B, 4), jnp.float32),
            pltpu.VMEM((B, 4), jnp.float32),
            pltpu.VMEM((B * 3,), jnp.float32),
            pltpu.VMEM((B * 3,), jnp.float32),
            pltpu.VMEM((B,), jnp.float32),
            pltpu.VMEM((B,), jnp.float32),
            pltpu.SemaphoreType.DMA,
            pltpu.SemaphoreType.DMA,
            pltpu.SemaphoreType.DMA,
            pltpu.SemaphoreType.DMA,
        ],
    )
    return kfn(pos4, edge_flat)


def kernel(pos, edge_index):
    pos4 = jnp.concatenate(
        [pos, jnp.zeros((N_NODES, 1), jnp.float32)], axis=1)
    edge_flat = edge_index.reshape(2 * N_EDGES)
    edge_weight, edge_vec_flat = _distance_sc(pos4, edge_flat)
    return (edge_index, edge_weight, edge_vec_flat.reshape(N_EDGES, 3))


# speedup vs baseline: 11.2355x; 1.0582x over previous
import jax
import jax.numpy as jnp
from jax import lax
from jax.experimental import pallas as pl
from jax.experimental.pallas import tpu as pltpu
from jax.experimental.pallas import tpu_sc as plsc

N_NODES = 100000
N_EDGES = 6400000

NW = 32
EPW = N_EDGES // NW
B = 2000
NCHUNK = EPW // B
NV = B // 16

_HALF = 0.5
_THREEHALF = 1.5
_MAGIC = 0x5F3759DF

_NSTAGE = 10
_SSLICE = N_NODES // _NSTAGE


def _norm16(dx, dy, dz):
    sq = dx * dx + dy * dy + dz * dz
    magic = jnp.full((16,), _MAGIC, jnp.int32)
    y = lax.bitcast_convert_type(
        magic - (lax.bitcast_convert_type(sq, jnp.int32) >> 1), jnp.float32)
    y = y * (_THREEHALF - _HALF * sq * y * y)
    y = y * (_THREEHALF - _HALF * sq * y * y)
    return jnp.where(sq > 0.0, sq * y, 0.0)


def _sc_body(pos_hbm, edge_hbm, w_hbm, vec_hbm,
             sx, sy, sz,
             si0, di0, si1, di1,
             g0, g1, vec0, vec1, w0, w1,
             stage_v, gsem0, gsem1, osem0, osem1):
    cid = lax.axis_index("c")
    sid = lax.axis_index("s")
    wid = cid * 16 + sid

    for c, comp in enumerate((sx, sy, sz)):
        for j in range(_NSTAGE):
            t = c * _NSTAGE + j

            @pl.when(sid == t % 16)
            def _(c=c, comp=comp, j=j):
                off = j * _SSLICE
                pltpu.sync_copy(
                    pos_hbm.at[pl.ds(c * N_NODES + off, _SSLICE)], stage_v)
                pltpu.sync_copy(stage_v, comp.at[pl.ds(off, _SSLICE)])

    plsc.subcore_barrier()

    lanes3 = 3 * lax.iota(jnp.int32, 16)
    bufs = ((si0, di0, g0, vec0, w0, gsem0, osem0),
            (si1, di1, g1, vec1, w1, gsem1, osem1))

    def load_idx(k, si, di):
        base = wid * EPW + k * B
        pltpu.sync_copy(edge_hbm.at[pl.ds(base, B)], si)
        pltpu.sync_copy(edge_hbm.at[pl.ds(N_EDGES + base, B)], di)

    def fire_gathers(si, di, g, gsem):
        for comp, dst in zip((sx, sy, sz), (g[0], g[1], g[2])):
            pltpu.async_copy(comp.at[si], dst, gsem)
        for comp, dst in zip((sx, sy, sz), (g[3], g[4], g[5])):
            pltpu.async_copy(comp.at[di], dst, gsem)

    def wait_gathers(si, di, g, gsem):
        for comp, dst in zip((sx, sy, sz), (g[0], g[1], g[2])):
            pltpu.make_async_copy(comp.at[si], dst, gsem).wait()
        for comp, dst in zip((sx, sy, sz), (g[3], g[4], g[5])):
            pltpu.make_async_copy(comp.at[di], dst, gsem).wait()

    def compute(g, vec_v, w_v):
        gsx, gsy, gsz, gdx, gdy, gdz = g

        def body(i, _):
            s = pl.ds(i * 16, 16)
            dx = gsx[s] - gdx[s]
            dy = gsy[s] - gdy[s]
            dz = gsz[s] - gdz[s]
            w_v[s] = _norm16(dx, dy, dz)
            flat = i * 48 + lanes3
            plsc.store_scatter(vec_v, [flat], dx)
            plsc.store_scatter(vec_v, [flat + 1], dy)
            plsc.store_scatter(vec_v, [flat + 2], dz)
            return 0

        lax.fori_loop(0, NV, body, 0)

    def fire_out(k, vec_v, w_v, osem):
        base = wid * EPW + k * B
        pltpu.async_copy(vec_v, vec_hbm.at[pl.ds(base * 3, B * 3)], osem)
        pltpu.async_copy(w_v, w_hbm.at[pl.ds(base, B)], osem)

    def wait_out(k, vec_v, w_v, osem):
        base = wid * EPW + k * B
        pltpu.make_async_copy(
            vec_v, vec_hbm.at[pl.ds(base * 3, B * 3)], osem).wait()
        pltpu.make_async_copy(w_v, w_hbm.at[pl.ds(base, B)], osem).wait()

    load_idx(0, si0, di0)
    fire_gathers(si0, di0, g0, gsem0)

    def outer(ki, _):
        for h in (0, 1):
            k = 2 * ki + h
            si, di, g, vec_v, w_v, gsem, osem = bufs[h]
            nsi, ndi, ng, _nv, _nw, ngsem, _no = bufs[1 - h]

            wait_gathers(si, di, g, gsem)

            @pl.when(k + 1 < NCHUNK)
            def _():
                load_idx(k + 1, nsi, ndi)
                fire_gathers(nsi, ndi, ng, ngsem)

            @pl.when(ki >= 1)
            def _():
                wait_out(k, vec_v, w_v, osem)

            compute(g, vec_v, w_v)
            fire_out(k, vec_v, w_v, osem)
        return 0

    lax.fori_loop(0, NCHUNK // 2, outer, 0)

    for h in (0, 1):
        _si, _di, _g, vec_v, w_v, _gs, osem = bufs[h]
        wait_out(0, vec_v, w_v, osem)


@jax.jit
def _distance_sc(pos_flat, edge_flat):
    mesh = plsc.VectorSubcoreMesh(core_axis_name="c", subcore_axis_name="s")
    kfn = pl.kernel(
        _sc_body,
        out_type=[
            jax.ShapeDtypeStruct((N_EDGES,), jnp.float32),
            jax.ShapeDtypeStruct((N_EDGES * 3,), jnp.float32),
        ],
        mesh=mesh,
        compiler_params=pltpu.CompilerParams(needs_layout_passes=False),
        scratch_types=[
            pltpu.VMEM_SHARED((N_NODES,), jnp.float32),
            pltpu.VMEM_SHARED((N_NODES,), jnp.float32),
            pltpu.VMEM_SHARED((N_NODES,), jnp.float32),
            pltpu.VMEM((B,), jnp.int32),
            pltpu.VMEM((B,), jnp.int32),
            pltpu.VMEM((B,), jnp.int32),
            pltpu.VMEM((B,), jnp.int32),
            [pltpu.VMEM((B,), jnp.float32)] * 6,
            [pltpu.VMEM((B,), jnp.float32)] * 6,
            pltpu.VMEM((B * 3,), jnp.float32),
            pltpu.VMEM((B * 3,), jnp.float32),
            pltpu.VMEM((B,), jnp.float32),
            pltpu.VMEM((B,), jnp.float32),
            pltpu.VMEM((_SSLICE,), jnp.float32),
            pltpu.SemaphoreType.DMA,
            pltpu.SemaphoreType.DMA,
            pltpu.SemaphoreType.DMA,
            pltpu.SemaphoreType.DMA,
        ],
    )
    return kfn(pos_flat, edge_flat)


def kernel(pos, edge_index):
    pos_flat = pos.T.reshape(3 * N_NODES)
    edge_flat = edge_index.reshape(2 * N_EDGES)
    edge_weight, edge_vec_flat = _distance_sc(pos_flat, edge_flat)
    return (edge_index, edge_weight, edge_vec_flat.reshape(N_EDGES, 3))


# speedup vs baseline: 89.0867x; 7.9290x over previous
import jax
import jax.numpy as jnp
from jax import lax
from jax.experimental import pallas as pl
from jax.experimental.pallas import tpu as pltpu
from jax.experimental.pallas import tpu_sc as plsc

N_NODES = 100000
N_EDGES = 6400000

NW = 32
NBLK = N_EDGES // 128
BLK_PW = NBLK // NW
NEXTRA = NBLK - NW * BLK_PW
CB = 11
NCHUNK = BLK_PW // CB
B = CB * 128
NG = B // 16

_HALF = 0.5
_THREEHALF = 1.5
_MAGIC = 0x5F3759DF

_NSTAGE = 10
_SSLICE = N_NODES // _NSTAGE


def _norm16(dx, dy, dz):
    sq = dx * dx + dy * dy + dz * dz
    magic = jnp.full((16,), _MAGIC, jnp.int32)
    y = lax.bitcast_convert_type(
        magic - (lax.bitcast_convert_type(sq, jnp.int32) >> 1), jnp.float32)
    y = y * (_THREEHALF - _HALF * sq * y * y)
    y = y * (_THREEHALF - _HALF * sq * y * y)
    return jnp.where(sq > 0.0, sq * y, 0.0)


def _compute(nblk, eb, gx, gy, gz, vec_v, w_v):
    def body(i, _):
        b = i // 8
        o = (i % 8) * 16
        s2 = pl.ds(b * 256 + o, 16)
        d2 = pl.ds(b * 256 + 128 + o, 16)
        dx = gx[s2] - gx[d2]
        dy = gy[s2] - gy[d2]
        dz = gz[s2] - gz[d2]
        w_v[pl.ds(b * 128 + o, 16)] = _norm16(dx, dy, dz)
        vec_v[pl.ds(b * 512 + o, 16)] = dx
        vec_v[pl.ds(b * 512 + 128 + o, 16)] = dy
        vec_v[pl.ds(b * 512 + 256 + o, 16)] = dz
        return 0

    lax.fori_loop(0, nblk * 8, body, 0)


def _sc_body(pos_hbm, edge_hbm, w_hbm, vec_hbm,
             sx, sy, sz,
             eb0, eb1, g0, g1, vec0, vec1, w0, w1,
             stage_v, gsem0, gsem1, osem0, osem1):
    cid = lax.axis_index("c")
    sid = lax.axis_index("s")
    wid = cid * 16 + sid

    for c, comp in enumerate((sx, sy, sz)):
        for j in range(_NSTAGE):
            t = c * _NSTAGE + j

            @pl.when(sid == t % 16)
            def _(c=c, comp=comp, j=j):
                off = j * _SSLICE
                pltpu.sync_copy(
                    pos_hbm.at[pl.ds(c * N_NODES + off, _SSLICE)], stage_v)
                pltpu.sync_copy(stage_v, comp.at[pl.ds(off, _SSLICE)])

    plsc.subcore_barrier()

    base_blk = wid * BLK_PW + jnp.minimum(wid, NEXTRA)
    bufs = ((eb0, g0[0], g0[1], g0[2], vec0, w0, gsem0, osem0),
            (eb1, g1[0], g1[1], g1[2], vec1, w1, gsem1, osem1))

    def load_idx(k, eb):
        off = (base_blk + k * CB) * 256
        pltpu.sync_copy(edge_hbm.at[pl.ds(off, CB * 256)], eb)

    def fire_gathers(eb, gx, gy, gz, gsem):
        for comp, dst in zip((sx, sy, sz), (gx, gy, gz)):
            pltpu.async_copy(comp.at[eb], dst, gsem)

    def wait_gathers(eb, gx, gy, gz, gsem):
        for comp, dst in zip((sx, sy, sz), (gx, gy, gz)):
            pltpu.make_async_copy(comp.at[eb], dst, gsem).wait()

    def fire_out(k, vec_v, w_v, osem):
        blk = base_blk + k * CB
        pltpu.async_copy(vec_v, vec_hbm.at[pl.ds(blk * 512, CB * 512)], osem)
        pltpu.async_copy(w_v, w_hbm.at[pl.ds(blk * 128, B)], osem)

    def wait_out(k, vec_v, w_v, osem):
        blk = base_blk + k * CB
        pltpu.make_async_copy(
            vec_v, vec_hbm.at[pl.ds(blk * 512, CB * 512)], osem).wait()
        pltpu.make_async_copy(
            w_v, w_hbm.at[pl.ds(blk * 128, B)], osem).wait()

    load_idx(0, eb0)
    fire_gathers(eb0, g0[0], g0[1], g0[2], gsem0)

    def outer(ki, _):
        for h in (0, 1):
            k = 2 * ki + h
            eb, gx, gy, gz, vec_v, w_v, gsem, osem = bufs[h]
            neb, ngx, ngy, ngz, _nv, _nw, ngsem, _no = bufs[1 - h]

            wait_gathers(eb, gx, gy, gz, gsem)

            @pl.when(k + 1 < NCHUNK)
            def _():
                load_idx(k + 1, neb)
                fire_gathers(neb, ngx, ngy, ngz, ngsem)

            @pl.when(ki >= 1)
            def _():
                wait_out(k, vec_v, w_v, osem)

            _compute(CB, eb, gx, gy, gz, vec_v, w_v)
            fire_out(k, vec_v, w_v, osem)
        return 0

    lax.fori_loop(0, NCHUNK // 2, outer, 0)

    for h in (0, 1):
        eb, gx, gy, gz, vec_v, w_v, gsem, osem = bufs[h]
        wait_out(0, vec_v, w_v, osem)

    @pl.when(wid < NEXTRA)
    def _():
        eb, gx, gy, gz, vec_v, w_v, gsem, osem = bufs[0]
        xblk = base_blk + BLK_PW
        pltpu.sync_copy(edge_hbm.at[pl.ds(xblk * 256, 256)],
                        eb.at[pl.ds(0, 256)])
        for comp, dst in zip((sx, sy, sz), (gx, gy, gz)):
            pltpu.sync_copy(comp.at[eb.at[pl.ds(0, 256)]],
                            dst.at[pl.ds(0, 256)])
        _compute(1, eb, gx, gy, gz, vec_v, w_v)
        pltpu.sync_copy(vec_v.at[pl.ds(0, 512)],
                        vec_hbm.at[pl.ds(xblk * 512, 512)])
        pltpu.sync_copy(w_v.at[pl.ds(0, 128)],
                        w_hbm.at[pl.ds(xblk * 128, 128)])


@jax.jit
def _distance_sc(pos_flat, edge_flat):
    mesh = plsc.VectorSubcoreMesh(core_axis_name="c", subcore_axis_name="s")
    kfn = pl.kernel(
        _sc_body,
        out_type=[
            jax.ShapeDtypeStruct((N_EDGES,), jnp.float32),
            jax.ShapeDtypeStruct((NBLK * 512,), jnp.float32),
        ],
        mesh=mesh,
        compiler_params=pltpu.CompilerParams(needs_layout_passes=False),
        scratch_types=[
            pltpu.VMEM_SHARED((N_NODES,), jnp.float32),
            pltpu.VMEM_SHARED((N_NODES,), jnp.float32),
            pltpu.VMEM_SHARED((N_NODES,), jnp.float32),
            pltpu.VMEM((CB * 256,), jnp.int32),
            pltpu.VMEM((CB * 256,), jnp.int32),
            [pltpu.VMEM((CB * 256,), jnp.float32)] * 3,
            [pltpu.VMEM((CB * 256,), jnp.float32)] * 3,
            pltpu.VMEM((CB * 512,), jnp.float32),
            pltpu.VMEM((CB * 512,), jnp.float32),
            pltpu.VMEM((B,), jnp.float32),
            pltpu.VMEM((B,), jnp.float32),
            pltpu.VMEM((_SSLICE,), jnp.float32),
            pltpu.SemaphoreType.DMA,
            pltpu.SemaphoreType.DMA,
            pltpu.SemaphoreType.DMA,
            pltpu.SemaphoreType.DMA,
        ],
    )
    return kfn(pos_flat, edge_flat)


def kernel(pos, edge_index):
    pos_flat = pos.T.reshape(3 * N_NODES)
    edge_flat = (edge_index.reshape(2, NBLK, 128)
                 .transpose(1, 0, 2).reshape(2 * N_EDGES))
    edge_weight, vec_flat = _distance_sc(pos_flat, edge_flat)
    edge_vec = (vec_flat.reshape(NBLK, 4, 128)[:, :3, :]
                .transpose(0, 2, 1).reshape(N_EDGES, 3))
    return (edge_index, edge_weight, edge_vec)


# speedup vs baseline: 89.1231x; 1.0004x over previous
import jax
import jax.numpy as jnp
from jax import lax
from jax.experimental import pallas as pl
from jax.experimental.pallas import tpu as pltpu
from jax.experimental.pallas import tpu_sc as plsc

N_NODES = 100000
N_EDGES = 6400000

NW = 32
NBLK = N_EDGES // 128
BLK_PW = NBLK // NW
NEXTRA = NBLK - NW * BLK_PW
CB = 11
NCHUNK = BLK_PW // CB
B = CB * 128
NG = B // 16

_HALF = 0.5
_THREEHALF = 1.5
_MAGIC = 0x5F3759DF

_NSTAGE = 10
_SSLICE = N_NODES // _NSTAGE


def _norm16(dx, dy, dz):
    sq = dx * dx + dy * dy + dz * dz
    sqm = jnp.maximum(sq, 1e-35)
    magic = jnp.full((16,), _MAGIC, jnp.int32)
    y = lax.bitcast_convert_type(
        magic - (lax.bitcast_convert_type(sqm, jnp.int32) >> 1), jnp.float32)
    y = y * (_THREEHALF - _HALF * sqm * y * y)
    y = y * (_THREEHALF - _HALF * sqm * y * y)
    return sq * y


def _compute(nblk, eb, gx, gy, gz, vec_v, w_v):
    def body(b, _):
        b256 = b * 256
        b128 = b * 128
        b512 = b * 512
        for g in range(8):
            o = g * 16
            s2 = pl.ds(b256 + o, 16)
            d2 = pl.ds(b256 + 128 + o, 16)
            dx = gx[s2] - gx[d2]
            dy = gy[s2] - gy[d2]
            dz = gz[s2] - gz[d2]
            w_v[pl.ds(b128 + o, 16)] = _norm16(dx, dy, dz)
            vec_v[pl.ds(b512 + o, 16)] = dx
            vec_v[pl.ds(b512 + 128 + o, 16)] = dy
            vec_v[pl.ds(b512 + 256 + o, 16)] = dz
        return 0

    lax.fori_loop(0, nblk, body, 0)


def _sc_body(pos_hbm, edge_hbm, w_hbm, vec_hbm,
             sx, sy, sz,
             eb0, eb1, g0, g1, vec0, vec1, w0, w1,
             stage_v, gsem0, gsem1, osem0, osem1):
    cid = lax.axis_index("c")
    sid = lax.axis_index("s")
    wid = cid * 16 + sid

    for c, comp in enumerate((sx, sy, sz)):
        for j in range(_NSTAGE):
            t = c * _NSTAGE + j

            @pl.when(sid == t % 16)
            def _(c=c, comp=comp, j=j):
                off = j * _SSLICE
                pltpu.sync_copy(
                    pos_hbm.at[pl.ds(c * N_NODES + off, _SSLICE)], stage_v)
                pltpu.sync_copy(stage_v, comp.at[pl.ds(off, _SSLICE)])

    plsc.subcore_barrier()

    base_blk = wid * BLK_PW + jnp.minimum(wid, NEXTRA)
    bufs = ((eb0, g0[0], g0[1], g0[2], vec0, w0, gsem0, osem0),
            (eb1, g1[0], g1[1], g1[2], vec1, w1, gsem1, osem1))

    def load_idx(k, eb):
        off = (base_blk + k * CB) * 256
        pltpu.sync_copy(edge_hbm.at[pl.ds(off, CB * 256)], eb)

    def fire_gathers(eb, gx, gy, gz, gsem):
        for comp, dst in zip((sx, sy, sz), (gx, gy, gz)):
            pltpu.async_copy(comp.at[eb], dst, gsem)

    def wait_gathers(eb, gx, gy, gz, gsem):
        for comp, dst in zip((sx, sy, sz), (gx, gy, gz)):
            pltpu.make_async_copy(comp.at[eb], dst, gsem).wait()

    def fire_out(k, vec_v, w_v, osem):
        blk = base_blk + k * CB
        pltpu.async_copy(vec_v, vec_hbm.at[pl.ds(blk * 512, CB * 512)], osem)
        pltpu.async_copy(w_v, w_hbm.at[pl.ds(blk * 128, B)], osem)

    def wait_out(k, vec_v, w_v, osem):
        blk = base_blk + k * CB
        pltpu.make_async_copy(
            vec_v, vec_hbm.at[pl.ds(blk * 512, CB * 512)], osem).wait()
        pltpu.make_async_copy(
            w_v, w_hbm.at[pl.ds(blk * 128, B)], osem).wait()

    load_idx(0, eb0)
    fire_gathers(eb0, g0[0], g0[1], g0[2], gsem0)

    def outer(ki, _):
        for h in (0, 1):
            k = 2 * ki + h
            eb, gx, gy, gz, vec_v, w_v, gsem, osem = bufs[h]
            neb, ngx, ngy, ngz, _nv, _nw, ngsem, _no = bufs[1 - h]

            wait_gathers(eb, gx, gy, gz, gsem)

            @pl.when(k + 1 < NCHUNK)
            def _():
                load_idx(k + 1, neb)
                fire_gathers(neb, ngx, ngy, ngz, ngsem)

            @pl.when(ki >= 1)
            def _():
                wait_out(k, vec_v, w_v, osem)

            _compute(CB, eb, gx, gy, gz, vec_v, w_v)
            fire_out(k, vec_v, w_v, osem)
        return 0

    lax.fori_loop(0, NCHUNK // 2, outer, 0)

    for h in (0, 1):
        eb, gx, gy, gz, vec_v, w_v, gsem, osem = bufs[h]
        wait_out(0, vec_v, w_v, osem)

    @pl.when(wid < NEXTRA)
    def _():
        eb, gx, gy, gz, vec_v, w_v, gsem, osem = bufs[0]
        xblk = base_blk + BLK_PW
        pltpu.sync_copy(edge_hbm.at[pl.ds(xblk * 256, 256)],
                        eb.at[pl.ds(0, 256)])
        for comp, dst in zip((sx, sy, sz), (gx, gy, gz)):
            pltpu.sync_copy(comp.at[eb.at[pl.ds(0, 256)]],
                            dst.at[pl.ds(0, 256)])
        _compute(1, eb, gx, gy, gz, vec_v, w_v)
        pltpu.sync_copy(vec_v.at[pl.ds(0, 512)],
                        vec_hbm.at[pl.ds(xblk * 512, 512)])
        pltpu.sync_copy(w_v.at[pl.ds(0, 128)],
                        w_hbm.at[pl.ds(xblk * 128, 128)])


@jax.jit
def _distance_sc(pos_flat, edge_flat):
    mesh = plsc.VectorSubcoreMesh(core_axis_name="c", subcore_axis_name="s")
    kfn = pl.kernel(
        _sc_body,
        out_type=[
            jax.ShapeDtypeStruct((N_EDGES,), jnp.float32),
            jax.ShapeDtypeStruct((NBLK * 512,), jnp.float32),
        ],
        mesh=mesh,
        compiler_params=pltpu.CompilerParams(needs_layout_passes=False),
        scratch_types=[
            pltpu.VMEM_SHARED((N_NODES,), jnp.float32),
            pltpu.VMEM_SHARED((N_NODES,), jnp.float32),
            pltpu.VMEM_SHARED((N_NODES,), jnp.float32),
            pltpu.VMEM((CB * 256,), jnp.int32),
            pltpu.VMEM((CB * 256,), jnp.int32),
            [pltpu.VMEM((CB * 256,), jnp.float32)] * 3,
            [pltpu.VMEM((CB * 256,), jnp.float32)] * 3,
            pltpu.VMEM((CB * 512,), jnp.float32),
            pltpu.VMEM((CB * 512,), jnp.float32),
            pltpu.VMEM((B,), jnp.float32),
            pltpu.VMEM((B,), jnp.float32),
            pltpu.VMEM((_SSLICE,), jnp.float32),
            pltpu.SemaphoreType.DMA,
            pltpu.SemaphoreType.DMA,
            pltpu.SemaphoreType.DMA,
            pltpu.SemaphoreType.DMA,
        ],
    )
    return kfn(pos_flat, edge_flat)


def kernel(pos, edge_index):
    pos_flat = pos.T.reshape(3 * N_NODES)
    edge_flat = (edge_index.reshape(2, NBLK, 128)
                 .transpose(1, 0, 2).reshape(2 * N_EDGES))
    edge_weight, vec_flat = _distance_sc(pos_flat, edge_flat)
    edge_vec = (vec_flat.reshape(NBLK, 4, 128)[:, :3, :]
                .transpose(0, 2, 1).reshape(N_EDGES, 3))
    return (edge_index, edge_weight, edge_vec)
